# baseline (speedup 1.0000x reference)
import functools
import math

import jax
import jax.numpy as jnp
from jax import lax
from jax.experimental import pallas as pl
from jax.experimental.pallas import tpu as pltpu


def _body(x_ref, sut_ref, out_ref, xs_ref, xs_s_ref, *, n_rows, n, c, k,
          n_iters):
    j = pl.program_id(1)
    inv_sqrt_c = 1.0 / math.sqrt(c)
    _m = math.isqrt(c)
    exact_scale = _m * _m == c and _m > 0 and (_m & (_m - 1)) == 0

    @pl.when(j == 0)
    def _():
        xs_ref[...] = jnp.sum(x_ref[0], axis=0)
        xs_s_ref[...] = (xs_ref[...] * inv_sqrt_c if exact_scale
                         else xs_ref[...])

    lhs = xs_ref[:, pl.ds(j * n_rows, n_rows)]
    a = lax.dot_general(lhs, xs_s_ref[...], (((0,), (0,)), ((), ())),
                        preferred_element_type=jnp.float32)
    if not exact_scale:
        a = a / math.sqrt(c)
    r = jnp.maximum(a, 0.0)
    e1 = jnp.exp(r - jnp.max(r, axis=-1, keepdims=True))
    s = e1 / jnp.sum(e1, axis=-1, keepdims=True)
    e2 = jnp.exp(s - jnp.max(s, axis=-1, keepdims=True))
    w = e2 / jnp.sum(e2, axis=-1, keepdims=True)

    bits = lax.bitcast_convert_type(w, jnp.int32)

    def prefix_count(eq):
        return lax.dot_general(eq.astype(jnp.bfloat16), sut_ref[...],
                               (((1,), (0,)), ((), ())),
                               preferred_element_type=jnp.float32)

    w_min = jnp.min(w, axis=-1, keepdims=True)
    t0 = lax.bitcast_convert_type(w_min, jnp.int32)
    eq0 = bits == t0
    pc0 = prefix_count(eq0)
    last = jnp.where(eq0[:, n - 1 : n], 1.0, 0.0)
    e0 = pc0[:, n - 1 : n] + last
    ok_fast = jnp.all(e0 > float(n - k))

    def fast_mask():
        return jnp.where((~eq0) | (pc0 < (k - n) + e0), w, 0.0)

    def search_mask():
        def search(i, carry):
            lo, hi = carry
            mid = (lo + hi) >> 1
            cnt = jnp.sum((bits >= mid).astype(jnp.int32), axis=-1,
                          keepdims=True)
            ge = cnt >= k
            return jnp.where(ge, mid, lo), jnp.where(ge, hi, mid)

        lo0 = jnp.zeros((n_rows, 1), jnp.int32)
        hi0 = jnp.full((n_rows, 1), 0x3F800001, jnp.int32)
        t = lax.fori_loop(0, n_iters, search, (lo0, hi0))[0]
        g = jnp.sum((bits > t).astype(jnp.int32), axis=-1, keepdims=True)
        eq = bits == t
        pc = prefix_count(eq)
        keep = (bits > t) | (eq & (pc < (k - g).astype(jnp.float32)))
        return jnp.where(keep, w, 0.0)

    out_ref[0] = lax.cond(ok_fast, fast_mask, search_mask)


def kernel(x):
    b, c, n, t = x.shape
    k = int(n * 0.8)
    n_rows = 512 if n % 512 == 0 else n
    xt = jnp.transpose(x, (0, 3, 1, 2))
    sut = (jnp.arange(n)[:, None] < jnp.arange(n)[None, :]).astype(jnp.bfloat16)
    body = functools.partial(_body, n_rows=n_rows, n=n, c=c, k=k, n_iters=30)
    return pl.pallas_call(
        body,
        grid=(b, n // n_rows),
        in_specs=[pl.BlockSpec((1, t, c, n), lambda bi, ji: (bi, 0, 0, 0)),
                  pl.BlockSpec((n, n), lambda bi, ji: (0, 0))],
        out_specs=pl.BlockSpec((1, n_rows, n), lambda bi, ji: (bi, ji, 0)),
        out_shape=jax.ShapeDtypeStruct((b, n, n), jnp.float32),
        scratch_shapes=[pltpu.VMEM((c, n), jnp.float32),
                        pltpu.VMEM((c, n), jnp.float32)],
    )(xt, sut)


# speedup vs baseline: 1.1998x; 1.1998x over previous
import functools
import math

import jax
import jax.numpy as jnp
from jax import lax
from jax.experimental import pallas as pl
from jax.experimental.pallas import tpu as pltpu


def _body(x_ref, sut_ref, out_ref, xs_ref, xs_s_ref, *, n_rows, n, c, k,
          n_iters):
    j = pl.program_id(1)
    inv_sqrt_c = 1.0 / math.sqrt(c)
    _m = math.isqrt(c)
    exact_scale = _m * _m == c and _m > 0 and (_m & (_m - 1)) == 0

    @pl.when(j == 0)
    def _():
        xs_ref[...] = jnp.sum(x_ref[0], axis=0)
        xs_s_ref[...] = (xs_ref[...] * inv_sqrt_c if exact_scale
                         else xs_ref[...])

    lhs = xs_ref[:, pl.ds(j * n_rows, n_rows)]
    a = lax.dot_general(lhs, xs_s_ref[...], (((0,), (0,)), ((), ())),
                        preferred_element_type=jnp.float32)
    if not exact_scale:
        a = a / math.sqrt(c)
    r = jnp.maximum(a, 0.0)
    e1 = jnp.exp(r - jnp.max(r, axis=-1, keepdims=True))
    s = e1 / jnp.sum(e1, axis=-1, keepdims=True)
    e2 = jnp.exp(s - jnp.max(s, axis=-1, keepdims=True))
    w = e2 / jnp.sum(e2, axis=-1, keepdims=True)

    bits = lax.bitcast_convert_type(w, jnp.int32)

    def prefix_count(eq):
        return lax.dot_general(eq.astype(jnp.bfloat16), sut_ref[...],
                               (((1,), (0,)), ((), ())),
                               preferred_element_type=jnp.float32)

    w_min = jnp.min(w, axis=-1, keepdims=True)
    t0 = lax.bitcast_convert_type(w_min, jnp.int32)
    gp = jnp.sum((bits > t0).astype(jnp.int32), axis=-1, keepdims=True)

    def full_search():
        def search(i, carry):
            lo, hi = carry
            mid = (lo + hi) >> 1
            cnt = jnp.sum((bits >= mid).astype(jnp.int32), axis=-1,
                          keepdims=True)
            ge = cnt >= k
            return jnp.where(ge, mid, lo), jnp.where(ge, hi, mid)

        lo0 = jnp.zeros((n_rows, 1), jnp.int32)
        hi0 = jnp.full((n_rows, 1), 0x3F800001, jnp.int32)
        t = lax.fori_loop(0, n_iters, search, (lo0, hi0))[0]
        g = jnp.sum((bits > t).astype(jnp.int32), axis=-1, keepdims=True)
        return t, g

    t, g = lax.cond(jnp.any(gp >= k), full_search, lambda: (t0, gp))

    gt = bits > t
    eq = bits == t
    pc = prefix_count(eq)
    keep = gt | (eq & (pc < (k - g).astype(jnp.float32)))
    out_ref[0] = jnp.where(keep, w, 0.0)


def kernel(x):
    b, c, n, t = x.shape
    k = int(n * 0.8)
    n_rows = 512 if n % 512 == 0 else n
    xt = jnp.transpose(x, (0, 3, 1, 2))
    sut = (jnp.arange(n)[:, None] < jnp.arange(n)[None, :]).astype(jnp.bfloat16)
    body = functools.partial(_body, n_rows=n_rows, n=n, c=c, k=k, n_iters=30)
    return pl.pallas_call(
        body,
        grid=(b, n // n_rows),
        in_specs=[pl.BlockSpec((1, t, c, n), lambda bi, ji: (bi, 0, 0, 0)),
                  pl.BlockSpec((n, n), lambda bi, ji: (0, 0))],
        out_specs=pl.BlockSpec((1, n_rows, n), lambda bi, ji: (bi, ji, 0)),
        out_shape=jax.ShapeDtypeStruct((b, n, n), jnp.float32),
        scratch_shapes=[pltpu.VMEM((c, n), jnp.float32),
                        pltpu.VMEM((c, n), jnp.float32)],
    )(xt, sut)


# speedup vs baseline: 1.2425x; 1.0356x over previous
import functools
import math

import jax
import jax.numpy as jnp
from jax import lax
from jax.experimental import pallas as pl
from jax.experimental.pallas import tpu as pltpu


def _body(x_ref, sut_ref, out_ref, xs_ref, xs_s_ref, *, n_rows, n, c, k,
          n_iters):
    j = pl.program_id(1)
    inv_sqrt_c = 1.0 / math.sqrt(c)
    _m = math.isqrt(c)
    exact_scale = _m * _m == c and _m > 0 and (_m & (_m - 1)) == 0

    @pl.when(j == 0)
    def _():
        xs_ref[...] = jnp.sum(x_ref[0], axis=0)
        xs_s_ref[...] = (xs_ref[...] * inv_sqrt_c if exact_scale
                         else xs_ref[...])

    lhs = xs_ref[:, pl.ds(j * n_rows, n_rows)]
    a = lax.dot_general(lhs, xs_s_ref[...], (((0,), (0,)), ((), ())),
                        preferred_element_type=jnp.float32)
    if not exact_scale:
        a = a / math.sqrt(c)
    r = jnp.maximum(a, 0.0)
    e1 = jnp.exp(r - jnp.max(r, axis=-1, keepdims=True))
    s = e1 / jnp.sum(e1, axis=-1, keepdims=True)
    e2 = jnp.exp(s - jnp.max(s, axis=-1, keepdims=True))
    w = e2 / jnp.sum(e2, axis=-1, keepdims=True)

    bits = lax.bitcast_convert_type(w, jnp.int32)

    def prefix_count(eq):
        return lax.dot_general(eq.astype(jnp.bfloat16), sut_ref[...],
                               (((1,), (0,)), ((), ())),
                               preferred_element_type=jnp.float32)

    w_min = jnp.min(w, axis=-1, keepdims=True)
    t0 = lax.bitcast_convert_type(w_min, jnp.int32)
    gp = jnp.sum((bits > t0).astype(jnp.int32), axis=-1, keepdims=True)

    def full_search():
        def search(i, carry):
            lo, hi = carry
            mid = (lo + hi) >> 1
            cnt = jnp.sum((bits >= mid).astype(jnp.int32), axis=-1,
                          keepdims=True)
            ge = cnt >= k
            return jnp.where(ge, mid, lo), jnp.where(ge, hi, mid)

        lo0 = jnp.zeros((n_rows, 1), jnp.int32)
        hi0 = jnp.full((n_rows, 1), 0x3F800001, jnp.int32)
        t = lax.fori_loop(0, n_iters, search, (lo0, hi0))[0]
        g = jnp.sum((bits > t).astype(jnp.int32), axis=-1, keepdims=True)
        return t, g

    t, g = lax.cond(jnp.any(gp >= k), full_search, lambda: (t0, gp))

    gt = bits > t
    eq = bits == t
    pc = prefix_count(eq)
    keep = gt | (eq & (pc < (k - g).astype(jnp.float32)))
    out_ref[0] = jnp.where(keep, w, 0.0)


def kernel(x):
    b, c, n, t = x.shape
    k = int(n * 0.8)
    n_rows = 1024 if n % 1024 == 0 else n
    xt = jnp.transpose(x, (0, 3, 1, 2))
    sut = (jnp.arange(n)[:, None] < jnp.arange(n)[None, :]).astype(jnp.bfloat16)
    body = functools.partial(_body, n_rows=n_rows, n=n, c=c, k=k, n_iters=30)
    return pl.pallas_call(
        body,
        grid=(b, n // n_rows),
        in_specs=[pl.BlockSpec((1, t, c, n), lambda bi, ji: (bi, 0, 0, 0)),
                  pl.BlockSpec((n, n), lambda bi, ji: (0, 0))],
        out_specs=pl.BlockSpec((1, n_rows, n), lambda bi, ji: (bi, ji, 0)),
        out_shape=jax.ShapeDtypeStruct((b, n, n), jnp.float32),
        scratch_shapes=[pltpu.VMEM((c, n), jnp.float32),
                        pltpu.VMEM((c, n), jnp.float32)],
    )(xt, sut)
